# Initial kernel scaffold; baseline (speedup 1.0000x reference)
import jax
import jax.numpy as jnp
from jax.experimental import pallas as pl


def kernel(x, edge_index, W1, b1, W2, b2, W3, b3, W4, b4, g2, be2, g3, be3, g4, be4):
    raise NotImplementedError("write your pallas kernel here")


# speedup vs baseline: 3.1124x; 3.1124x over previous
import jax
import jax.numpy as jnp
from jax import lax
from jax.experimental import pallas as pl
from jax.experimental.pallas import tpu as pltpu
from jax.experimental.pallas import tpu_sc as plsc

N = 10000
E = 160000
B = 16
CIN = 128
HID = 256
COUT = 10
K = 4
NP = 10240
BNP = B * NP
BN_ROWS = B * N
CC = 128
HC = HID // CC

NT = 16
NC = 2
EPT = E // NT
EB = 128
TAIL = EPT - (EPT // EB) * EB
BLOCKS = [(i * EB, EB) for i in range(EPT // EB)] + [((EPT // EB) * EB, TAIL)]
RPT = NP // NT
DR = 64
NDR = RPT // DR


def _mesh():
    return plsc.VectorSubcoreMesh(core_axis_name="c", subcore_axis_name="s")


def _deg_body(e32, out, ids, ones, zb, db, acc):
    cid = lax.axis_index("c")
    tid = lax.axis_index("s")
    one16 = jnp.ones((16,), jnp.float32)
    zero16 = jnp.zeros((16,), jnp.float32)

    def fill(i, _):
        ones[pl.ds(i * 16, 16)] = one16
        return 0
    lax.fori_loop(0, 400 // 16, fill, 0)

    def fillz(i, _):
        zb[pl.ds(i * 16, 16)] = zero16
        return 0
    lax.fori_loop(0, 640 // 16, fillz, 0)

    pltpu.sync_copy(zb, acc.at[pl.ds(tid * 640, 640)])
    plsc.subcore_barrier()
    for j in range(EPT // 400):
        off = cid * E + tid * EPT + j * 400
        pltpu.sync_copy(e32.at[pl.ds(off, 400)], ids)
        pltpu.sync_copy(ones, acc.at[ids], add=True)
    plsc.subcore_barrier()
    pltpu.sync_copy(acc.at[pl.ds(tid * 640, 640)], db)
    pltpu.sync_copy(db, out.at[pl.ds(cid * NP + tid * 640, 640)])


@jax.jit
def _degrees(e32):
    return pl.kernel(
        _deg_body,
        out_type=jax.ShapeDtypeStruct((NC * NP,), jnp.float32),
        mesh=_mesh(),
        scratch_types=[
            pltpu.VMEM((400,), jnp.int32),
            pltpu.VMEM((400,), jnp.float32),
            pltpu.VMEM((640,), jnp.float32),
            pltpu.VMEM((640,), jnp.float32),
            pltpu.VMEM_SHARED((NP,), jnp.float32),
        ],
    )(e32)


def _sval(s_t, idx):
    return s_t[pl.ds(idx, 16)][0]


def _gpass(src_view, acc, soff, e32, cid, tid, bufs, ib, sems):
    isrc0, isrc1, isrc_t, idst, idst_t = ib
    isrcs = (isrc0, isrc1)
    idx_base = cid * E + tid * EPT

    def _gather(blk, p):
        off, sz = BLOCKS[blk]
        iref = isrcs[p] if sz == EB else isrc_t
        pltpu.sync_copy(soff.at[pl.ds(idx_base + off, sz)], iref)
        dst = bufs[p] if sz == EB else bufs[p].at[pl.ds(0, TAIL)]
        return pltpu.async_copy(src_view.at[iref], dst, sems[p])

    d_cur = _gather(0, 0)
    for j in range(len(BLOCKS)):
        d_nxt = None
        if j + 1 < len(BLOCKS):
            d_nxt = _gather(j + 1, (j + 1) % 2)
        d_cur.wait()
        off, sz = BLOCKS[j]
        dref = idst if sz == EB else idst_t
        pltpu.sync_copy(e32.at[pl.ds(E + tid * EPT + off, sz)], dref)
        srcb = bufs[j % 2] if sz == EB else bufs[j % 2].at[pl.ds(0, TAIL)]
        pltpu.sync_copy(srcb, acc.at[dref], add=True)
        d_cur = d_nxt


def _cheb_body(xh, e32, soff, st, t1, t2, t3, sS,
               isrc0, isrc1, isrc_t, idst, idst_t, r0b, r1b, s_t,
               sem0, sem1, acc):
    NCHC = (B * HC) // NC
    cid = lax.axis_index("c")
    tid = lax.axis_index("s")
    r0 = tid * RPT
    bufs = (r0b, r1b)
    ib = (isrc0, isrc1, isrc_t, idst, idst_t)
    sems = (sem0, sem1)
    dbuf_v, sclb_v = r0b.at[pl.ds(0, DR)], r0b.at[pl.ds(DR, DR)]
    xbuf_v, zbuf_v = r1b.at[pl.ds(0, DR)], r1b.at[pl.ds(DR, DR)]
    sW = cid * NP

    pltpu.sync_copy(st.at[pl.ds(tid * 640, 640)], s_t)

    def refill_zeros():
        zero16 = jnp.zeros((16,), jnp.float32)

        def body(r, _):
            for w in range(CC // 16):
                r1b[DR + r, pl.ds(w * 16, 16)] = zero16
            return 0
        lax.fori_loop(0, DR, body, 0)

    def chunk_body(ci, _):
        ch = ci * NC + cid
        b = ch // HC
        h = ch % HC
        rowbase = b * NP + r0
        col = h * CC

        refill_zeros()
        for i in range(NDR):
            pltpu.sync_copy(
                xh.at[pl.ds(rowbase + i * DR, DR), pl.ds(col, CC)], xbuf_v)

            def s0(r, _):
                sv = _sval(s_t, i * DR + r)
                for w in range(CC // 16):
                    r0b[DR + r, pl.ds(w * 16, 16)] = (
                        sv * r1b[r, pl.ds(w * 16, 16)])
                return 0
            lax.fori_loop(0, DR, s0, 0)
            pltpu.sync_copy(sclb_v, sS.at[pl.ds(sW + r0 + i * DR, DR)])
            pltpu.sync_copy(zbuf_v, acc.at[pl.ds(r0 + i * DR, DR)])
        plsc.subcore_barrier()

        for stage in (1, 2, 3):
            _gpass(sS, acc, soff, e32, cid, tid, bufs, ib, sems)
            plsc.subcore_barrier()
            out_ref = (t1, t2, t3)[stage - 1]
            refill_zeros()
            for i in range(NDR):
                lsl = pl.ds(r0 + i * DR, DR)
                hsl = pl.ds(rowbase + i * DR, DR)
                csl = pl.ds(col, CC)
                pltpu.sync_copy(acc.at[lsl], dbuf_v)
                pltpu.sync_copy(zbuf_v, acc.at[lsl])
                if stage == 2:
                    pltpu.sync_copy(xh.at[hsl, csl], xbuf_v)
                elif stage == 3:
                    pltpu.sync_copy(t1.at[hsl, csl], xbuf_v)

                def combine(r, _):
                    sv = _sval(s_t, i * DR + r)
                    for w in range(CC // 16):
                        ws = pl.ds(w * 16, 16)
                        a = r0b[r, ws]
                        if stage == 1:
                            tv = -a
                        else:
                            tv = -2.0 * a - r1b[r, ws]
                        r0b[r, ws] = tv
                        if stage < 3:
                            r0b[DR + r, ws] = sv * tv
                    return 0
                lax.fori_loop(0, DR, combine, 0)
                pltpu.sync_copy(dbuf_v, out_ref.at[hsl, csl])
                if stage < 3:
                    pltpu.sync_copy(sclb_v, sS.at[pl.ds(sW + r0 + i * DR, DR)])
            plsc.subcore_barrier()
        return 0

    lax.fori_loop(0, NCHC, chunk_body, 0)


def _make_cheb():
    sds = jax.ShapeDtypeStruct((BNP, HID), jnp.float32)
    return pl.kernel(
        _cheb_body,
        out_type=(sds, sds, sds,
                  jax.ShapeDtypeStruct((NC * NP, CC), jnp.float32)),
        mesh=_mesh(),
        scratch_types=[
            pltpu.VMEM((EB,), jnp.int32),
            pltpu.VMEM((EB,), jnp.int32),
            pltpu.VMEM((TAIL,), jnp.int32),
            pltpu.VMEM((EB,), jnp.int32),
            pltpu.VMEM((TAIL,), jnp.int32),
            pltpu.VMEM((EB, CC), jnp.float32),
            pltpu.VMEM((EB, CC), jnp.float32),
            pltpu.VMEM((640,), jnp.float32),
            pltpu.SemaphoreType.DMA,
            pltpu.SemaphoreType.DMA,
            pltpu.VMEM_SHARED((NP, CC), jnp.float32),
        ],
    )


def _mm4_body(t0, t1, t2, t3, w, bias, vcol, out):
    acc = jnp.dot(t0[...], w[0], preferred_element_type=jnp.float32)
    acc += jnp.dot(t1[...], w[1], preferred_element_type=jnp.float32)
    acc += jnp.dot(t2[...], w[2], preferred_element_type=jnp.float32)
    acc += jnp.dot(t3[...], w[3], preferred_element_type=jnp.float32)
    y = jnp.maximum(acc * vcol[...] + bias[...], 0.0)
    i = pl.program_id(0)
    r = y.shape[0]
    rid = (i % (NP // r)) * r + lax.broadcasted_iota(jnp.int32, y.shape, 0)
    out[...] = jnp.where(rid < N, y, 0.0)


def _mm4(t0, t1, t2, t3, w, bias, vcol, O):
    R = 640
    C = w.shape[1]
    tspec = pl.BlockSpec((R, C), lambda i: (i, 0))
    return pl.pallas_call(
        _mm4_body,
        grid=(BNP // R,),
        in_specs=[
            tspec, tspec, tspec, tspec,
            pl.BlockSpec((K, C, O), lambda i: (0, 0, 0)),
            pl.BlockSpec((1, O), lambda i: (0, 0)),
            pl.BlockSpec((R, 1), lambda i: (i, 0)),
        ],
        out_specs=pl.BlockSpec((R, O), lambda i: (i, 0)),
        out_shape=jax.ShapeDtypeStruct((BNP, O), jnp.float32),
    )(t0, t1, t2, t3, w, bias.reshape(1, O), vcol)


_cheb = _make_cheb()


def kernel(x, edge_index, W1, b1, W2, b2, W3, b3, W4, b4,
           g2, be2, g3, be3, g4, be4):
    e32 = edge_index.reshape(2 * E)
    soff = jnp.concatenate([edge_index[0], edge_index[0] + NP])

    degs = _degrees(e32)
    deg_out = degs[:N]
    deg_in = degs[NP:NP + N]
    u = jnp.where(deg_out > 0, lax.rsqrt(jnp.maximum(deg_out, 1.0)), 0.0)
    v = jnp.where(deg_in > 0, lax.rsqrt(jnp.maximum(deg_in, 1.0)), 1.0)
    s = u * v
    invv = jnp.where(deg_in > 0, jnp.sqrt(jnp.maximum(deg_in, 1.0)), 1.0)

    s_tiles = jnp.pad(s, (0, NP - N))
    v_pad = jnp.pad(v, (0, NP - N), constant_values=1.0)
    invv_pad = jnp.pad(invv, (0, NP - N), constant_values=1.0)
    v_rows = jnp.tile(v_pad, B)[:, None]
    invv_rows = jnp.tile(invv_pad, B)[:, None]

    def bn_prescale(y, g, be):
        m = jnp.sum(y, axis=0) / BN_ROWS
        var = jnp.sum(y * y, axis=0) / BN_ROWS - m * m
        alpha = g * lax.rsqrt(var + 1e-5)
        beta = be - m * alpha
        return (y * alpha[None, :] + beta[None, :]) * invv_rows

    xt = jnp.pad(x.transpose(0, 2, 1),
                 ((0, 0), (0, NP - N), (0, HID - CIN)))
    xh = xt.reshape(BNP, HID) * invv_rows
    w1p = jnp.pad(W1, ((0, 0), (0, HID - CIN), (0, 0)))
    t1, t2, t3, _ = _cheb(xh, e32, soff, s_tiles)
    y = _mm4(xh, t1, t2, t3, w1p, b1, v_rows, HID)

    xh = bn_prescale(y, g2, be2)
    t1, t2, t3, _ = _cheb(xh, e32, soff, s_tiles)
    y = _mm4(xh, t1, t2, t3, W2, b2, v_rows, HID)

    xh = bn_prescale(y, g3, be3)
    t1, t2, t3, _ = _cheb(xh, e32, soff, s_tiles)
    y = _mm4(xh, t1, t2, t3, W3, b3, v_rows, HID)

    xh = bn_prescale(y, g4, be4)
    w4p = jnp.pad(W4, ((0, 0), (0, 0), (0, 16 - COUT)))
    b4p = jnp.pad(b4, (0, 16 - COUT))
    t1, t2, t3, _ = _cheb(xh, e32, soff, s_tiles)
    y4 = _mm4(xh, t1, t2, t3, w4p, b4p, v_rows, 16)

    pooled = y4.reshape(B, NP, 16)[:, :N, :COUT].mean(axis=1)
    return jax.nn.log_softmax(pooled, axis=1)


# speedup vs baseline: 3.8549x; 1.2386x over previous
import jax
import jax.numpy as jnp
from jax import lax
from jax.experimental import pallas as pl
from jax.experimental.pallas import tpu as pltpu
from jax.experimental.pallas import tpu_sc as plsc

N = 10000
E = 160000
B = 16
CIN = 128
HID = 256
COUT = 10
K = 4
NP = 10240
BNP = B * NP
BN_ROWS = B * N
CC = 128
HC = HID // CC

NT = 16
NC = 2
EPT = E // NT
EB = 160
TAIL = EPT - (EPT // EB) * EB
BLOCKS = [(i * EB, EB) for i in range(EPT // EB)] + [((EPT // EB) * EB, TAIL)]
RPT = NP // NT
DR = 64
NDR = RPT // DR


def _mesh():
    return plsc.VectorSubcoreMesh(core_axis_name="c", subcore_axis_name="s")


def _deg_body(e32, out, ids, ones, zb, db, acc):
    cid = lax.axis_index("c")
    tid = lax.axis_index("s")
    one16 = jnp.ones((16,), jnp.float32)
    zero16 = jnp.zeros((16,), jnp.float32)

    def fill(i, _):
        ones[pl.ds(i * 16, 16)] = one16
        return 0
    lax.fori_loop(0, 400 // 16, fill, 0)

    def fillz(i, _):
        zb[pl.ds(i * 16, 16)] = zero16
        return 0
    lax.fori_loop(0, 640 // 16, fillz, 0)

    pltpu.sync_copy(zb, acc.at[pl.ds(tid * 640, 640)])
    plsc.subcore_barrier()
    for j in range(EPT // 400):
        off = cid * E + tid * EPT + j * 400
        pltpu.sync_copy(e32.at[pl.ds(off, 400)], ids)
        pltpu.sync_copy(ones, acc.at[ids], add=True)
    plsc.subcore_barrier()
    pltpu.sync_copy(acc.at[pl.ds(tid * 640, 640)], db)
    pltpu.sync_copy(db, out.at[pl.ds(cid * NP + tid * 640, 640)])


@jax.jit
def _degrees(e32):
    return pl.kernel(
        _deg_body,
        out_type=jax.ShapeDtypeStruct((NC * NP,), jnp.float32),
        mesh=_mesh(),
        scratch_types=[
            pltpu.VMEM((400,), jnp.int32),
            pltpu.VMEM((400,), jnp.float32),
            pltpu.VMEM((640,), jnp.float32),
            pltpu.VMEM((640,), jnp.float32),
            pltpu.VMEM_SHARED((NP,), jnp.float32),
        ],
    )(e32)


def _sval(s_t, idx):
    return s_t[pl.ds(idx, 16)][0]


def _gpass(src_view, acc, soff, e32, cid, tid, bufs, ib, sems):
    isrc0, isrc1, isrc_t, idst, idst_t = ib
    isrcs = (isrc0, isrc1)
    idx_base = cid * E + tid * EPT

    def _gather(blk, p):
        off, sz = BLOCKS[blk]
        iref = isrcs[p] if sz == EB else isrc_t
        pltpu.sync_copy(soff.at[pl.ds(idx_base + off, sz)], iref)
        dst = bufs[p] if sz == EB else bufs[p].at[pl.ds(0, TAIL)]
        return pltpu.async_copy(src_view.at[iref], dst, sems[p])

    d_cur = _gather(0, 0)
    for j in range(len(BLOCKS)):
        d_nxt = None
        if j + 1 < len(BLOCKS):
            d_nxt = _gather(j + 1, (j + 1) % 2)
        d_cur.wait()
        off, sz = BLOCKS[j]
        dref = idst if sz == EB else idst_t
        pltpu.sync_copy(e32.at[pl.ds(E + tid * EPT + off, sz)], dref)
        srcb = bufs[j % 2] if sz == EB else bufs[j % 2].at[pl.ds(0, TAIL)]
        pltpu.sync_copy(srcb, acc.at[dref], add=True)
        d_cur = d_nxt


def _cheb_body(xh, e32, soff, st, ncnt, t1, t2, t3, sS,
               isrc0, isrc1, isrc_t, idst, idst_t, r0b, r1b, s_t, nbuf,
               sem0, sem1, acc):
    cid = lax.axis_index("c")
    tid = lax.axis_index("s")
    r0 = tid * RPT
    bufs = (r0b, r1b)
    ib = (isrc0, isrc1, isrc_t, idst, idst_t)
    sems = (sem0, sem1)
    dbuf_v, sclb_v = r0b.at[pl.ds(0, DR)], r0b.at[pl.ds(DR, DR)]
    xbuf_v, zbuf_v = r1b.at[pl.ds(0, DR)], r1b.at[pl.ds(DR, DR)]
    sW = cid * NP

    pltpu.sync_copy(st.at[pl.ds(tid * 640, 640)], s_t)
    pltpu.sync_copy(ncnt.at[pl.ds(0, 16)], nbuf)
    nchc = nbuf[pl.ds(0, 16)][0]

    def refill_zeros():
        zero16 = jnp.zeros((16,), jnp.float32)

        def body(r, _):
            for w in range(CC // 16):
                r1b[DR + r, pl.ds(w * 16, 16)] = zero16
            return 0
        lax.fori_loop(0, DR, body, 0)

    def chunk_body(ci, _):
        ch = ci * NC + cid
        h = ch // B
        b = ch % B
        rowbase = b * NP + r0
        col = h * CC

        refill_zeros()
        for i in range(NDR):
            pltpu.sync_copy(
                xh.at[pl.ds(rowbase + i * DR, DR), pl.ds(col, CC)], xbuf_v)

            def s0(r, _):
                sv = _sval(s_t, i * DR + r)
                for w in range(CC // 16):
                    r0b[DR + r, pl.ds(w * 16, 16)] = (
                        sv * r1b[r, pl.ds(w * 16, 16)])
                return 0
            lax.fori_loop(0, DR, s0, 0)
            pltpu.sync_copy(sclb_v, sS.at[pl.ds(sW + r0 + i * DR, DR)])
            pltpu.sync_copy(zbuf_v, acc.at[pl.ds(r0 + i * DR, DR)])
        plsc.subcore_barrier()

        for stage in (1, 2, 3):
            _gpass(sS, acc, soff, e32, cid, tid, bufs, ib, sems)
            plsc.subcore_barrier()
            out_ref = (t1, t2, t3)[stage - 1]
            refill_zeros()
            for i in range(NDR):
                lsl = pl.ds(r0 + i * DR, DR)
                hsl = pl.ds(rowbase + i * DR, DR)
                csl = pl.ds(col, CC)
                pltpu.sync_copy(acc.at[lsl], dbuf_v)
                pltpu.sync_copy(zbuf_v, acc.at[lsl])
                if stage == 2:
                    pltpu.sync_copy(xh.at[hsl, csl], xbuf_v)
                elif stage == 3:
                    pltpu.sync_copy(t1.at[hsl, csl], xbuf_v)

                def combine(r, _):
                    sv = _sval(s_t, i * DR + r)
                    for w in range(CC // 16):
                        ws = pl.ds(w * 16, 16)
                        a = r0b[r, ws]
                        if stage == 1:
                            tv = -a
                        else:
                            tv = -2.0 * a - r1b[r, ws]
                        r0b[r, ws] = tv
                        if stage < 3:
                            r0b[DR + r, ws] = sv * tv
                    return 0
                lax.fori_loop(0, DR, combine, 0)
                pltpu.sync_copy(dbuf_v, out_ref.at[hsl, csl])
                if stage < 3:
                    pltpu.sync_copy(sclb_v, sS.at[pl.ds(sW + r0 + i * DR, DR)])
            plsc.subcore_barrier()
        return 0

    lax.fori_loop(0, nchc, chunk_body, 0)


def _make_cheb():
    sds = jax.ShapeDtypeStruct((BNP, HID), jnp.float32)
    return pl.kernel(
        _cheb_body,
        out_type=(sds, sds, sds,
                  jax.ShapeDtypeStruct((NC * NP, CC), jnp.float32)),
        mesh=_mesh(),
        scratch_types=[
            pltpu.VMEM((EB,), jnp.int32),
            pltpu.VMEM((EB,), jnp.int32),
            pltpu.VMEM((TAIL,), jnp.int32),
            pltpu.VMEM((EB,), jnp.int32),
            pltpu.VMEM((TAIL,), jnp.int32),
            pltpu.VMEM((EB, CC), jnp.float32),
            pltpu.VMEM((EB, CC), jnp.float32),
            pltpu.VMEM((640,), jnp.float32),
            pltpu.VMEM((16,), jnp.int32),
            pltpu.SemaphoreType.DMA,
            pltpu.SemaphoreType.DMA,
            pltpu.VMEM_SHARED((NP, CC), jnp.float32),
        ],
    )


def _mm4_body(t0, t1, t2, t3, w, bias, vcol, out):
    acc = jnp.dot(t0[...], w[0], preferred_element_type=jnp.float32)
    acc += jnp.dot(t1[...], w[1], preferred_element_type=jnp.float32)
    acc += jnp.dot(t2[...], w[2], preferred_element_type=jnp.float32)
    acc += jnp.dot(t3[...], w[3], preferred_element_type=jnp.float32)
    y = jnp.maximum(acc * vcol[...] + bias[...], 0.0)
    i = pl.program_id(0)
    r = y.shape[0]
    rid = (i % (NP // r)) * r + lax.broadcasted_iota(jnp.int32, y.shape, 0)
    out[...] = jnp.where(rid < N, y, 0.0)


def _mm4(t0, t1, t2, t3, w, bias, vcol, O):
    R = 640
    C = w.shape[1]
    tspec = pl.BlockSpec((R, C), lambda i: (i, 0))
    return pl.pallas_call(
        _mm4_body,
        grid=(BNP // R,),
        in_specs=[
            tspec, tspec, tspec, tspec,
            pl.BlockSpec((K, C, O), lambda i: (0, 0, 0)),
            pl.BlockSpec((1, O), lambda i: (0, 0)),
            pl.BlockSpec((R, 1), lambda i: (i, 0)),
        ],
        out_specs=pl.BlockSpec((R, O), lambda i: (i, 0)),
        out_shape=jax.ShapeDtypeStruct((BNP, O), jnp.float32),
    )(t0, t1, t2, t3, w, bias.reshape(1, O), vcol)


_cheb = _make_cheb()


def kernel(x, edge_index, W1, b1, W2, b2, W3, b3, W4, b4,
           g2, be2, g3, be3, g4, be4):
    e32 = edge_index.reshape(2 * E)
    soff = jnp.concatenate([edge_index[0], edge_index[0] + NP])

    degs = _degrees(e32)
    deg_out = degs[:N]
    deg_in = degs[NP:NP + N]
    u = jnp.where(deg_out > 0, lax.rsqrt(jnp.maximum(deg_out, 1.0)), 0.0)
    v = jnp.where(deg_in > 0, lax.rsqrt(jnp.maximum(deg_in, 1.0)), 1.0)
    s = u * v
    invv = jnp.where(deg_in > 0, jnp.sqrt(jnp.maximum(deg_in, 1.0)), 1.0)

    s_tiles = jnp.pad(s, (0, NP - N))
    nc_half = jnp.full((16,), (B * 1) // NC, jnp.int32)
    nc_full = jnp.full((16,), (B * HC) // NC, jnp.int32)
    v_pad = jnp.pad(v, (0, NP - N), constant_values=1.0)
    invv_pad = jnp.pad(invv, (0, NP - N), constant_values=1.0)
    v_rows = jnp.tile(v_pad, B)[:, None]
    invv_rows = jnp.tile(invv_pad, B)[:, None]

    def bn_prescale(y, g, be):
        m = jnp.sum(y, axis=0) / BN_ROWS
        var = jnp.sum(y * y, axis=0) / BN_ROWS - m * m
        alpha = g * lax.rsqrt(var + 1e-5)
        beta = be - m * alpha
        return (y * alpha[None, :] + beta[None, :]) * invv_rows

    xt = jnp.pad(x.transpose(0, 2, 1),
                 ((0, 0), (0, NP - N), (0, HID - CIN)))
    xh = xt.reshape(BNP, HID) * invv_rows
    t1, t2, t3, _ = _cheb(xh, e32, soff, s_tiles, nc_half)
    y = _mm4(xh, t1, t2, t3, W1, b1, v_rows, HID)

    xh = bn_prescale(y, g2, be2)
    t1, t2, t3, _ = _cheb(xh, e32, soff, s_tiles, nc_full)
    y = _mm4(xh, t1, t2, t3, W2, b2, v_rows, HID)

    xh = bn_prescale(y, g3, be3)
    t1, t2, t3, _ = _cheb(xh, e32, soff, s_tiles, nc_full)
    y = _mm4(xh, t1, t2, t3, W3, b3, v_rows, HID)

    xh = bn_prescale(y, g4, be4)
    w4p = jnp.pad(W4, ((0, 0), (0, 0), (0, 16 - COUT)))
    b4p = jnp.pad(b4, (0, 16 - COUT))
    t1, t2, t3, _ = _cheb(xh, e32, soff, s_tiles, nc_full)
    y4 = _mm4(xh, t1, t2, t3, w4p, b4p, v_rows, 16)

    pooled = y4.reshape(B, NP, 16)[:, :N, :COUT].mean(axis=1)
    return jax.nn.log_softmax(pooled, axis=1)


# speedup vs baseline: 3.8605x; 1.0015x over previous
import jax
import jax.numpy as jnp
from jax import lax
from jax.experimental import pallas as pl
from jax.experimental.pallas import tpu as pltpu
from jax.experimental.pallas import tpu_sc as plsc

N = 10000
E = 160000
B = 16
CIN = 128
HID = 256
COUT = 10
K = 4
NP = 10240
BNP = B * NP
BN_ROWS = B * N
CC = 128
HC = HID // CC

NT = 16
NC = 2
EPT = E // NT
EB = 160
TAIL = EPT - (EPT // EB) * EB
BLOCKS = [(i * EB, EB) for i in range(EPT // EB)] + [((EPT // EB) * EB, TAIL)]
RPT = NP // NT
DR = 64
NDR = RPT // DR


def _mesh():
    return plsc.VectorSubcoreMesh(core_axis_name="c", subcore_axis_name="s")


def _deg_body(e32, out, ids, ones, zb, db, acc):
    cid = lax.axis_index("c")
    tid = lax.axis_index("s")
    one16 = jnp.ones((16,), jnp.float32)
    zero16 = jnp.zeros((16,), jnp.float32)

    def fill(i, _):
        ones[pl.ds(i * 16, 16)] = one16
        return 0
    lax.fori_loop(0, 400 // 16, fill, 0)

    def fillz(i, _):
        zb[pl.ds(i * 16, 16)] = zero16
        return 0
    lax.fori_loop(0, 640 // 16, fillz, 0)

    pltpu.sync_copy(zb, acc.at[pl.ds(tid * 640, 640)])
    plsc.subcore_barrier()
    for j in range(EPT // 400):
        off = cid * E + tid * EPT + j * 400
        pltpu.sync_copy(e32.at[pl.ds(off, 400)], ids)
        pltpu.sync_copy(ones, acc.at[ids], add=True)
    plsc.subcore_barrier()
    pltpu.sync_copy(acc.at[pl.ds(tid * 640, 640)], db)
    pltpu.sync_copy(db, out.at[pl.ds(cid * NP + tid * 640, 640)])


@jax.jit
def _degrees(e32):
    return pl.kernel(
        _deg_body,
        out_type=jax.ShapeDtypeStruct((NC * NP,), jnp.float32),
        mesh=_mesh(),
        scratch_types=[
            pltpu.VMEM((400,), jnp.int32),
            pltpu.VMEM((400,), jnp.float32),
            pltpu.VMEM((640,), jnp.float32),
            pltpu.VMEM((640,), jnp.float32),
            pltpu.VMEM_SHARED((NP,), jnp.float32),
        ],
    )(e32)


def _sval(s_t, idx):
    return s_t[pl.ds(idx, 16)][0]


def _gpass(src_view, acc, soff, e32, cid, tid, bufs, ib, sems):
    isrc0, isrc1, isrc_t, idst0, idst1, idst_t = ib
    isrcs = (isrc0, isrc1)
    idsts = (idst0, idst1)
    gsems, ssems = sems
    idx_base = cid * E + tid * EPT
    nb = len(BLOCKS)

    def _gather(blk, p):
        off, sz = BLOCKS[blk]
        iref = isrcs[p] if sz == EB else isrc_t
        pltpu.sync_copy(soff.at[pl.ds(idx_base + off, sz)], iref)
        dst = bufs[p] if sz == EB else bufs[p].at[pl.ds(0, TAIL)]
        return pltpu.async_copy(src_view.at[iref], dst, gsems[p])

    def _scatter(blk, p):
        off, sz = BLOCKS[blk]
        dref = idsts[p] if sz == EB else idst_t
        pltpu.sync_copy(e32.at[pl.ds(E + tid * EPT + off, sz)], dref)
        srcb = bufs[p] if sz == EB else bufs[p].at[pl.ds(0, TAIL)]
        return pltpu.async_copy(srcb, acc.at[dref], ssems[p], add=True)

    d_g = [None, None]
    d_s = [None, None]
    d_g[0] = _gather(0, 0)
    for j in range(nb):
        p = j % 2
        if j + 1 < nb:
            if d_s[1 - p] is not None:
                d_s[1 - p].wait()
                d_s[1 - p] = None
            d_g[1 - p] = _gather(j + 1, 1 - p)
        d_g[p].wait()
        d_s[p] = _scatter(j, p)
    for p in (0, 1):
        if d_s[p] is not None:
            d_s[p].wait()


def _cheb_body(xh, e32, soff, st, ncnt, t1, t2, t3, sS,
               isrc0, isrc1, isrc_t, idst0, idst1, idst_t, r0b, r1b, s_t,
               nbuf, sem0, sem1, sem2, sem3, acc):
    cid = lax.axis_index("c")
    tid = lax.axis_index("s")
    r0 = tid * RPT
    bufs = (r0b, r1b)
    ib = (isrc0, isrc1, isrc_t, idst0, idst1, idst_t)
    sems = ((sem0, sem1), (sem2, sem3))
    dbuf_v, sclb_v = r0b.at[pl.ds(0, DR)], r0b.at[pl.ds(DR, DR)]
    xbuf_v, zbuf_v = r1b.at[pl.ds(0, DR)], r1b.at[pl.ds(DR, DR)]
    sW = cid * NP

    pltpu.sync_copy(st.at[pl.ds(tid * 640, 640)], s_t)
    pltpu.sync_copy(ncnt.at[pl.ds(0, 16)], nbuf)
    nchc = nbuf[pl.ds(0, 16)][0]

    def refill_zeros():
        zero16 = jnp.zeros((16,), jnp.float32)

        def body(r, _):
            for w in range(CC // 16):
                r1b[DR + r, pl.ds(w * 16, 16)] = zero16
            return 0
        lax.fori_loop(0, DR, body, 0)

    def chunk_body(ci, _):
        ch = ci * NC + cid
        h = ch // B
        b = ch % B
        rowbase = b * NP + r0
        col = h * CC

        refill_zeros()
        for i in range(NDR):
            pltpu.sync_copy(
                xh.at[pl.ds(rowbase + i * DR, DR), pl.ds(col, CC)], xbuf_v)

            def s0(r, _):
                sv = _sval(s_t, i * DR + r)
                for w in range(CC // 16):
                    r0b[DR + r, pl.ds(w * 16, 16)] = (
                        sv * r1b[r, pl.ds(w * 16, 16)])
                return 0
            lax.fori_loop(0, DR, s0, 0)
            pltpu.sync_copy(sclb_v, sS.at[pl.ds(sW + r0 + i * DR, DR)])
            pltpu.sync_copy(zbuf_v, acc.at[pl.ds(r0 + i * DR, DR)])
        plsc.subcore_barrier()

        for stage in (1, 2, 3):
            _gpass(sS, acc, soff, e32, cid, tid, bufs, ib, sems)
            plsc.subcore_barrier()
            out_ref = (t1, t2, t3)[stage - 1]
            refill_zeros()
            for i in range(NDR):
                lsl = pl.ds(r0 + i * DR, DR)
                hsl = pl.ds(rowbase + i * DR, DR)
                csl = pl.ds(col, CC)
                pltpu.sync_copy(acc.at[lsl], dbuf_v)
                pltpu.sync_copy(zbuf_v, acc.at[lsl])
                if stage == 2:
                    pltpu.sync_copy(xh.at[hsl, csl], xbuf_v)
                elif stage == 3:
                    pltpu.sync_copy(t1.at[hsl, csl], xbuf_v)

                def combine(r, _):
                    sv = _sval(s_t, i * DR + r)
                    for w in range(CC // 16):
                        ws = pl.ds(w * 16, 16)
                        a = r0b[r, ws]
                        if stage == 1:
                            tv = -a
                        else:
                            tv = -2.0 * a - r1b[r, ws]
                        r0b[r, ws] = tv
                        if stage < 3:
                            r0b[DR + r, ws] = sv * tv
                    return 0
                lax.fori_loop(0, DR, combine, 0)
                pltpu.sync_copy(dbuf_v, out_ref.at[hsl, csl])
                if stage < 3:
                    pltpu.sync_copy(sclb_v, sS.at[pl.ds(sW + r0 + i * DR, DR)])
            plsc.subcore_barrier()
        return 0

    lax.fori_loop(0, nchc, chunk_body, 0)


def _make_cheb():
    sds = jax.ShapeDtypeStruct((BNP, HID), jnp.float32)
    return pl.kernel(
        _cheb_body,
        out_type=(sds, sds, sds,
                  jax.ShapeDtypeStruct((NC * NP, CC), jnp.float32)),
        mesh=_mesh(),
        scratch_types=[
            pltpu.VMEM((EB,), jnp.int32),
            pltpu.VMEM((EB,), jnp.int32),
            pltpu.VMEM((TAIL,), jnp.int32),
            pltpu.VMEM((EB,), jnp.int32),
            pltpu.VMEM((EB,), jnp.int32),
            pltpu.VMEM((TAIL,), jnp.int32),
            pltpu.VMEM((EB, CC), jnp.float32),
            pltpu.VMEM((EB, CC), jnp.float32),
            pltpu.VMEM((640,), jnp.float32),
            pltpu.VMEM((16,), jnp.int32),
            pltpu.SemaphoreType.DMA,
            pltpu.SemaphoreType.DMA,
            pltpu.SemaphoreType.DMA,
            pltpu.SemaphoreType.DMA,
            pltpu.VMEM_SHARED((NP, CC), jnp.float32),
        ],
    )


def _mm4_body(t0, t1, t2, t3, w, bias, vcol, out):
    acc = jnp.dot(t0[...], w[0], preferred_element_type=jnp.float32)
    acc += jnp.dot(t1[...], w[1], preferred_element_type=jnp.float32)
    acc += jnp.dot(t2[...], w[2], preferred_element_type=jnp.float32)
    acc += jnp.dot(t3[...], w[3], preferred_element_type=jnp.float32)
    y = jnp.maximum(acc * vcol[...] + bias[...], 0.0)
    i = pl.program_id(0)
    r = y.shape[0]
    rid = (i % (NP // r)) * r + lax.broadcasted_iota(jnp.int32, y.shape, 0)
    out[...] = jnp.where(rid < N, y, 0.0)


def _mm4(t0, t1, t2, t3, w, bias, vcol, O):
    R = 640
    C = w.shape[1]
    tspec = pl.BlockSpec((R, C), lambda i: (i, 0))
    return pl.pallas_call(
        _mm4_body,
        grid=(BNP // R,),
        in_specs=[
            tspec, tspec, tspec, tspec,
            pl.BlockSpec((K, C, O), lambda i: (0, 0, 0)),
            pl.BlockSpec((1, O), lambda i: (0, 0)),
            pl.BlockSpec((R, 1), lambda i: (i, 0)),
        ],
        out_specs=pl.BlockSpec((R, O), lambda i: (i, 0)),
        out_shape=jax.ShapeDtypeStruct((BNP, O), jnp.float32),
    )(t0, t1, t2, t3, w, bias.reshape(1, O), vcol)


_cheb = _make_cheb()


def kernel(x, edge_index, W1, b1, W2, b2, W3, b3, W4, b4,
           g2, be2, g3, be3, g4, be4):
    e32 = edge_index.reshape(2 * E)
    soff = jnp.concatenate([edge_index[0], edge_index[0] + NP])

    degs = _degrees(e32)
    deg_out = degs[:N]
    deg_in = degs[NP:NP + N]
    u = jnp.where(deg_out > 0, lax.rsqrt(jnp.maximum(deg_out, 1.0)), 0.0)
    v = jnp.where(deg_in > 0, lax.rsqrt(jnp.maximum(deg_in, 1.0)), 1.0)
    s = u * v
    invv = jnp.where(deg_in > 0, jnp.sqrt(jnp.maximum(deg_in, 1.0)), 1.0)

    s_tiles = jnp.pad(s, (0, NP - N))
    nc_half = jnp.full((16,), (B * 1) // NC, jnp.int32)
    nc_full = jnp.full((16,), (B * HC) // NC, jnp.int32)
    v_pad = jnp.pad(v, (0, NP - N), constant_values=1.0)
    invv_pad = jnp.pad(invv, (0, NP - N), constant_values=1.0)
    v_rows = jnp.tile(v_pad, B)[:, None]
    invv_rows = jnp.tile(invv_pad, B)[:, None]

    def bn_prescale(y, g, be):
        m = jnp.sum(y, axis=0) / BN_ROWS
        var = jnp.sum(y * y, axis=0) / BN_ROWS - m * m
        alpha = g * lax.rsqrt(var + 1e-5)
        beta = be - m * alpha
        return (y * alpha[None, :] + beta[None, :]) * invv_rows

    xt = jnp.pad(x.transpose(0, 2, 1),
                 ((0, 0), (0, NP - N), (0, HID - CIN)))
    xh = xt.reshape(BNP, HID) * invv_rows
    t1, t2, t3, _ = _cheb(xh, e32, soff, s_tiles, nc_half)
    y = _mm4(xh, t1, t2, t3, W1, b1, v_rows, HID)

    xh = bn_prescale(y, g2, be2)
    t1, t2, t3, _ = _cheb(xh, e32, soff, s_tiles, nc_full)
    y = _mm4(xh, t1, t2, t3, W2, b2, v_rows, HID)

    xh = bn_prescale(y, g3, be3)
    t1, t2, t3, _ = _cheb(xh, e32, soff, s_tiles, nc_full)
    y = _mm4(xh, t1, t2, t3, W3, b3, v_rows, HID)

    xh = bn_prescale(y, g4, be4)
    w4p = jnp.pad(W4, ((0, 0), (0, 0), (0, 16 - COUT)))
    b4p = jnp.pad(b4, (0, 16 - COUT))
    t1, t2, t3, _ = _cheb(xh, e32, soff, s_tiles, nc_full)
    y4 = _mm4(xh, t1, t2, t3, w4p, b4p, v_rows, 16)

    pooled = y4.reshape(B, NP, 16)[:, :N, :COUT].mean(axis=1)
    return jax.nn.log_softmax(pooled, axis=1)


# speedup vs baseline: 4.0560x; 1.0506x over previous
import jax
import jax.numpy as jnp
from jax import lax
from jax.experimental import pallas as pl
from jax.experimental.pallas import tpu as pltpu
from jax.experimental.pallas import tpu_sc as plsc

N = 10000
E = 160000
B = 16
CIN = 128
HID = 256
COUT = 10
K = 4
NP = 10240
BNP = B * NP
BN_ROWS = B * N
CC = 128
HC = HID // CC

NT = 16
NC = 2
EPT = E // NT
EB = 160
TAIL = EPT - (EPT // EB) * EB
BLOCKS = [(i * EB, EB) for i in range(EPT // EB)] + [((EPT // EB) * EB, TAIL)]
RPT = NP // NT
DR = 64
NDR = RPT // DR


def _mesh():
    return plsc.VectorSubcoreMesh(core_axis_name="c", subcore_axis_name="s")


def _deg_body(e32, out, ids, ones, zb, db, acc):
    cid = lax.axis_index("c")
    tid = lax.axis_index("s")
    one16 = jnp.ones((16,), jnp.float32)
    zero16 = jnp.zeros((16,), jnp.float32)

    def fill(i, _):
        ones[pl.ds(i * 16, 16)] = one16
        return 0
    lax.fori_loop(0, 400 // 16, fill, 0)

    def fillz(i, _):
        zb[pl.ds(i * 16, 16)] = zero16
        return 0
    lax.fori_loop(0, 640 // 16, fillz, 0)

    pltpu.sync_copy(zb, acc.at[pl.ds(tid * 640, 640)])
    plsc.subcore_barrier()
    for j in range(EPT // 400):
        off = cid * E + tid * EPT + j * 400
        pltpu.sync_copy(e32.at[pl.ds(off, 400)], ids)
        pltpu.sync_copy(ones, acc.at[ids], add=True)
    plsc.subcore_barrier()
    pltpu.sync_copy(acc.at[pl.ds(tid * 640, 640)], db)
    pltpu.sync_copy(db, out.at[pl.ds(cid * NP + tid * 640, 640)])


@jax.jit
def _degrees(e32):
    return pl.kernel(
        _deg_body,
        out_type=jax.ShapeDtypeStruct((NC * NP,), jnp.float32),
        mesh=_mesh(),
        scratch_types=[
            pltpu.VMEM((400,), jnp.int32),
            pltpu.VMEM((400,), jnp.float32),
            pltpu.VMEM((640,), jnp.float32),
            pltpu.VMEM((640,), jnp.float32),
            pltpu.VMEM_SHARED((NP,), jnp.float32),
        ],
    )(e32)


def _sval(s_t, idx):
    return s_t[pl.ds(idx, 16)][0]


def _gpass(src_view, acc, soff, e32, cid, tid, bufs, ib, sems):
    isrc0, isrc1, isrc_t, idst0, idst1, idst_t = ib
    isrcs = (isrc0, isrc1)
    idsts = (idst0, idst1)
    gsems, ssems = sems
    idx_base = cid * E + tid * EPT
    nb = len(BLOCKS)

    def _gather(blk, p):
        off, sz = BLOCKS[blk]
        iref = isrcs[p] if sz == EB else isrc_t
        pltpu.sync_copy(soff.at[pl.ds(idx_base + off, sz)], iref)
        dst = bufs[p] if sz == EB else bufs[p].at[pl.ds(0, TAIL)]
        return pltpu.async_copy(src_view.at[iref], dst, gsems[p])

    def _scatter(blk, p):
        off, sz = BLOCKS[blk]
        dref = idsts[p] if sz == EB else idst_t
        pltpu.sync_copy(e32.at[pl.ds(E + tid * EPT + off, sz)], dref)
        srcb = bufs[p] if sz == EB else bufs[p].at[pl.ds(0, TAIL)]
        return pltpu.async_copy(srcb, acc.at[dref], ssems[p], add=True)

    d_g = [None, None]
    d_s = [None, None]
    d_g[0] = _gather(0, 0)
    for j in range(nb):
        p = j % 2
        if j + 1 < nb:
            if d_s[1 - p] is not None:
                d_s[1 - p].wait()
                d_s[1 - p] = None
            d_g[1 - p] = _gather(j + 1, 1 - p)
        d_g[p].wait()
        d_s[p] = _scatter(j, p)
    for p in (0, 1):
        if d_s[p] is not None:
            d_s[p].wait()


def _cheb_body(xh, e32, soff, st, ncnt, t1, t2, t3, sS,
               isrc0, isrc1, isrc_t, idst0, idst1, idst_t, r0b, r1b, s_t,
               nbuf, sem0, sem1, sem2, sem3, acc):
    cid = lax.axis_index("c")
    tid = lax.axis_index("s")
    r0 = tid * RPT
    bufs = (r0b, r1b)
    ib = (isrc0, isrc1, isrc_t, idst0, idst1, idst_t)
    sems = ((sem0, sem1), (sem2, sem3))
    dbuf_v, sclb_v = r0b.at[pl.ds(0, DR)], r0b.at[pl.ds(DR, DR)]
    xbuf_v, zbuf_v = r1b.at[pl.ds(0, DR)], r1b.at[pl.ds(DR, DR)]
    sW = cid * NP

    pltpu.sync_copy(st.at[pl.ds(tid * 640, 640)], s_t)
    pltpu.sync_copy(ncnt.at[pl.ds(0, 16)], nbuf)
    nchc = nbuf[pl.ds(0, 16)][0]

    def refill_zeros():
        zero16 = jnp.zeros((16,), jnp.float32)

        def body(r, _):
            for w in range(CC // 16):
                r1b[DR + r, pl.ds(w * 16, 16)] = zero16
            return 0
        lax.fori_loop(0, DR, body, 0)

    def chunk_body(ci, _):
        ch = ci * NC + cid
        h = ch // B
        b = ch % B
        rowbase = b * NP + r0
        col = h * CC

        refill_zeros()
        d_s = d_z = None
        for i in range(NDR):
            d_b = pltpu.async_copy(
                xh.at[pl.ds(rowbase + i * DR, DR), pl.ds(col, CC)],
                xbuf_v, sem1)
            if d_s is not None:
                d_s.wait()
            d_b.wait()

            def s0(r, _):
                sv = _sval(s_t, i * DR + r)
                for w in range(CC // 16):
                    r0b[DR + r, pl.ds(w * 16, 16)] = (
                        sv * r1b[r, pl.ds(w * 16, 16)])
                return 0
            lax.fori_loop(0, DR, s0, 0)
            d_s = pltpu.async_copy(
                sclb_v, sS.at[pl.ds(sW + r0 + i * DR, DR)], sem0)
            if d_z is not None:
                d_z.wait()
            d_z = pltpu.async_copy(
                zbuf_v, acc.at[pl.ds(r0 + i * DR, DR)], sem2)
        d_s.wait()
        d_z.wait()
        plsc.subcore_barrier()

        for stage in (1, 2, 3):
            _gpass(sS, acc, soff, e32, cid, tid, bufs, ib, sems)
            plsc.subcore_barrier()
            out_ref = (t1, t2, t3)[stage - 1]
            refill_zeros()
            d_o = d_s = d_z = None
            for i in range(NDR):
                lsl = pl.ds(r0 + i * DR, DR)
                hsl = pl.ds(rowbase + i * DR, DR)
                csl = pl.ds(col, CC)
                if d_o is not None:
                    d_o.wait()
                if d_s is not None:
                    d_s.wait()
                if d_z is not None:
                    d_z.wait()
                d_a = pltpu.async_copy(acc.at[lsl], dbuf_v, sem0)
                d_b = None
                if stage == 2:
                    d_b = pltpu.async_copy(xh.at[hsl, csl], xbuf_v, sem1)
                elif stage == 3:
                    d_b = pltpu.async_copy(t1.at[hsl, csl], xbuf_v, sem1)
                d_a.wait()
                d_z = pltpu.async_copy(zbuf_v, acc.at[lsl], sem2)
                if d_b is not None:
                    d_b.wait()

                def combine(r, _):
                    sv = _sval(s_t, i * DR + r)
                    for w in range(CC // 16):
                        ws = pl.ds(w * 16, 16)
                        a = r0b[r, ws]
                        if stage == 1:
                            tv = -a
                        else:
                            tv = -2.0 * a - r1b[r, ws]
                        r0b[r, ws] = tv
                        if stage < 3:
                            r0b[DR + r, ws] = sv * tv
                    return 0
                lax.fori_loop(0, DR, combine, 0)
                d_o = pltpu.async_copy(dbuf_v, out_ref.at[hsl, csl], sem3)
                d_s = None
                if stage < 3:
                    d_s = pltpu.async_copy(
                        sclb_v, sS.at[pl.ds(sW + r0 + i * DR, DR)], sem0)
            d_o.wait()
            if d_s is not None:
                d_s.wait()
            d_z.wait()
            plsc.subcore_barrier()
        return 0

    lax.fori_loop(0, nchc, chunk_body, 0)


def _make_cheb():
    sds = jax.ShapeDtypeStruct((BNP, HID), jnp.float32)
    return pl.kernel(
        _cheb_body,
        out_type=(sds, sds, sds,
                  jax.ShapeDtypeStruct((NC * NP, CC), jnp.float32)),
        mesh=_mesh(),
        scratch_types=[
            pltpu.VMEM((EB,), jnp.int32),
            pltpu.VMEM((EB,), jnp.int32),
            pltpu.VMEM((TAIL,), jnp.int32),
            pltpu.VMEM((EB,), jnp.int32),
            pltpu.VMEM((EB,), jnp.int32),
            pltpu.VMEM((TAIL,), jnp.int32),
            pltpu.VMEM((EB, CC), jnp.float32),
            pltpu.VMEM((EB, CC), jnp.float32),
            pltpu.VMEM((640,), jnp.float32),
            pltpu.VMEM((16,), jnp.int32),
            pltpu.SemaphoreType.DMA,
            pltpu.SemaphoreType.DMA,
            pltpu.SemaphoreType.DMA,
            pltpu.SemaphoreType.DMA,
            pltpu.VMEM_SHARED((NP, CC), jnp.float32),
        ],
    )


def _mm4_body(t0, t1, t2, t3, w, bias, vcol, out):
    acc = jnp.dot(t0[...], w[0], preferred_element_type=jnp.float32)
    acc += jnp.dot(t1[...], w[1], preferred_element_type=jnp.float32)
    acc += jnp.dot(t2[...], w[2], preferred_element_type=jnp.float32)
    acc += jnp.dot(t3[...], w[3], preferred_element_type=jnp.float32)
    y = jnp.maximum(acc * vcol[...] + bias[...], 0.0)
    i = pl.program_id(0)
    r = y.shape[0]
    rid = (i % (NP // r)) * r + lax.broadcasted_iota(jnp.int32, y.shape, 0)
    out[...] = jnp.where(rid < N, y, 0.0)


def _mm4(t0, t1, t2, t3, w, bias, vcol, O):
    R = 640
    C = w.shape[1]
    tspec = pl.BlockSpec((R, C), lambda i: (i, 0))
    return pl.pallas_call(
        _mm4_body,
        grid=(BNP // R,),
        in_specs=[
            tspec, tspec, tspec, tspec,
            pl.BlockSpec((K, C, O), lambda i: (0, 0, 0)),
            pl.BlockSpec((1, O), lambda i: (0, 0)),
            pl.BlockSpec((R, 1), lambda i: (i, 0)),
        ],
        out_specs=pl.BlockSpec((R, O), lambda i: (i, 0)),
        out_shape=jax.ShapeDtypeStruct((BNP, O), jnp.float32),
    )(t0, t1, t2, t3, w, bias.reshape(1, O), vcol)


_cheb = _make_cheb()


def kernel(x, edge_index, W1, b1, W2, b2, W3, b3, W4, b4,
           g2, be2, g3, be3, g4, be4):
    e32 = edge_index.reshape(2 * E)
    soff = jnp.concatenate([edge_index[0], edge_index[0] + NP])

    degs = _degrees(e32)
    deg_out = degs[:N]
    deg_in = degs[NP:NP + N]
    u = jnp.where(deg_out > 0, lax.rsqrt(jnp.maximum(deg_out, 1.0)), 0.0)
    v = jnp.where(deg_in > 0, lax.rsqrt(jnp.maximum(deg_in, 1.0)), 1.0)
    s = u * v
    invv = jnp.where(deg_in > 0, jnp.sqrt(jnp.maximum(deg_in, 1.0)), 1.0)

    s_tiles = jnp.pad(s, (0, NP - N))
    nc_half = jnp.full((16,), (B * 1) // NC, jnp.int32)
    nc_full = jnp.full((16,), (B * HC) // NC, jnp.int32)
    v_pad = jnp.pad(v, (0, NP - N), constant_values=1.0)
    invv_pad = jnp.pad(invv, (0, NP - N), constant_values=1.0)
    v_rows = jnp.tile(v_pad, B)[:, None]
    invv_rows = jnp.tile(invv_pad, B)[:, None]

    def bn_prescale(y, g, be):
        m = jnp.sum(y, axis=0) / BN_ROWS
        var = jnp.sum(y * y, axis=0) / BN_ROWS - m * m
        alpha = g * lax.rsqrt(var + 1e-5)
        beta = be - m * alpha
        return (y * alpha[None, :] + beta[None, :]) * invv_rows

    xt = jnp.pad(x.transpose(0, 2, 1),
                 ((0, 0), (0, NP - N), (0, HID - CIN)))
    xh = xt.reshape(BNP, HID) * invv_rows
    t1, t2, t3, _ = _cheb(xh, e32, soff, s_tiles, nc_half)
    y = _mm4(xh, t1, t2, t3, W1, b1, v_rows, HID)

    xh = bn_prescale(y, g2, be2)
    t1, t2, t3, _ = _cheb(xh, e32, soff, s_tiles, nc_full)
    y = _mm4(xh, t1, t2, t3, W2, b2, v_rows, HID)

    xh = bn_prescale(y, g3, be3)
    t1, t2, t3, _ = _cheb(xh, e32, soff, s_tiles, nc_full)
    y = _mm4(xh, t1, t2, t3, W3, b3, v_rows, HID)

    xh = bn_prescale(y, g4, be4)
    w4p = jnp.pad(W4, ((0, 0), (0, 0), (0, 16 - COUT)))
    b4p = jnp.pad(b4, (0, 16 - COUT))
    t1, t2, t3, _ = _cheb(xh, e32, soff, s_tiles, nc_full)
    y4 = _mm4(xh, t1, t2, t3, w4p, b4p, v_rows, 16)

    pooled = y4.reshape(B, NP, 16)[:, :N, :COUT].mean(axis=1)
    return jax.nn.log_softmax(pooled, axis=1)
